# baseline (speedup 1.0000x reference)
import functools

import jax
import jax.numpy as jnp
from jax import lax
from jax.experimental import pallas as pl
from jax.experimental.pallas import tpu as pltpu
from jax.experimental.pallas import tpu_sc as plsc

N_ENTITIES = 1000000
N_RELATIONS = 1000
EMBED_DIM = 64
BATCH = 16384
WIDE = 2 * EMBED_DIM

NUM_CORES = 2
NUM_SUBCORES = 16
NUM_WORKERS = NUM_CORES * NUM_SUBCORES
B_PER_W = BATCH // NUM_WORKERS
LANES = 16
CHUNK = 128
N_CHUNKS = B_PER_W // CHUNK
GROUPS_PER_CHUNK = CHUNK // LANES


def _neg_sqrt(x):
    i = lax.bitcast_convert_type(x, jnp.int32)
    i = 0x5F3759DF - lax.shift_right_arithmetic(i, 1)
    y = lax.bitcast_convert_type(i, jnp.float32)
    half_x = 0.5 * x
    y = y * (1.5 - half_x * y * y)
    y = y * (1.5 - half_x * y * y)
    y = y * (1.5 - half_x * y * y)
    return -(x * y)


def _sc_body(h_hbm, t_hbm, r_hbm, ent_hbm, rel_hbm, out_hbm,
             idx_h, idx_t, idx_r, tix_h, tix_t, tix_r,
             rows_h, rows_t, rows_r, out_v, sem):
    wid = lax.axis_index("s") * NUM_CORES + lax.axis_index("c")
    base = wid * B_PER_W

    pltpu.sync_copy(h_hbm.at[pl.ds(base, B_PER_W)], idx_h)
    pltpu.sync_copy(t_hbm.at[pl.ds(base, B_PER_W)], idx_t)
    pltpu.sync_copy(r_hbm.at[pl.ds(base, B_PER_W)], idx_r)

    lane = lax.iota(jnp.int32, LANES)

    def tix_body(g, _):
        sl = pl.ds(g * LANES, LANES)
        tix_h[sl] = lax.shift_right_logical(idx_h[sl], 1)
        tix_t[sl] = lax.shift_right_logical(idx_t[sl], 1)
        tix_r[sl] = lax.shift_right_logical(idx_r[sl], 1)
        return _

    lax.fori_loop(0, B_PER_W // LANES, tix_body, 0)

    def chunk_body(c, _):
        csl = pl.ds(c * CHUNK, CHUNK)
        cp_h = pltpu.async_copy(ent_hbm.at[tix_h.at[csl]], rows_h, sem)
        cp_t = pltpu.async_copy(ent_hbm.at[tix_t.at[csl]], rows_t, sem)
        cp_r = pltpu.async_copy(rel_hbm.at[tix_r.at[csl]], rows_r, sem)
        cp_h.wait()
        cp_t.wait()
        cp_r.wait()

        def group_body(g, _):
            k_vec = g * LANES + lane
            gsl = pl.ds(c * CHUNK + g * LANES, LANES)
            off_h = lax.bitwise_and(idx_h[gsl], 1) * EMBED_DIM
            off_t = lax.bitwise_and(idx_t[gsl], 1) * EMBED_DIM
            off_r = lax.bitwise_and(idx_r[gsl], 1) * EMBED_DIM
            acc = jnp.zeros((LANES,), jnp.float32)

            def dim_body(d, acc):
                vh = plsc.load_gather(rows_h, [k_vec, off_h + d])
                vr = plsc.load_gather(rows_r, [k_vec, off_r + d])
                vt = plsc.load_gather(rows_t, [k_vec, off_t + d])
                df = (vh + vr) - vt
                return acc + df * df

            acc = lax.fori_loop(0, EMBED_DIM, dim_body, acc)
            out_v[gsl] = _neg_sqrt(acc + 1e-12)
            return _

        lax.fori_loop(0, GROUPS_PER_CHUNK, group_body, 0)
        return _

    lax.fori_loop(0, N_CHUNKS, chunk_body, 0)

    pltpu.sync_copy(out_v, out_hbm.at[pl.ds(base, B_PER_W)])


@jax.jit
def _score(h, t, r, entity_emb, relation_emb):
    ent2 = entity_emb.reshape(N_ENTITIES // 2, WIDE)
    rel2 = relation_emb.reshape(N_RELATIONS // 2, WIDE)
    mesh = plsc.VectorSubcoreMesh(core_axis_name="c", subcore_axis_name="s")
    run = functools.partial(
        pl.kernel,
        mesh=mesh,
        compiler_params=pltpu.CompilerParams(needs_layout_passes=False),
        out_type=jax.ShapeDtypeStruct((BATCH,), jnp.float32),
        scratch_types=[
            pltpu.VMEM((B_PER_W,), jnp.int32),
            pltpu.VMEM((B_PER_W,), jnp.int32),
            pltpu.VMEM((B_PER_W,), jnp.int32),
            pltpu.VMEM((B_PER_W,), jnp.int32),
            pltpu.VMEM((B_PER_W,), jnp.int32),
            pltpu.VMEM((B_PER_W,), jnp.int32),
            pltpu.VMEM((CHUNK, WIDE), jnp.float32),
            pltpu.VMEM((CHUNK, WIDE), jnp.float32),
            pltpu.VMEM((CHUNK, WIDE), jnp.float32),
            pltpu.VMEM((B_PER_W,), jnp.float32),
            pltpu.SemaphoreType.DMA,
        ],
    )(_sc_body)
    return run(h, t, r, ent2, rel2)


def kernel(h, t, r, entity_emb, relation_emb):
    return _score(h.astype(jnp.int32), t.astype(jnp.int32),
                  r.astype(jnp.int32), entity_emb, relation_emb)
